# baseline (speedup 1.0000x reference)
import functools

import jax
import jax.numpy as jnp
from jax import lax
from jax.experimental import pallas as pl
from jax.experimental.pallas import tpu as pltpu
from jax.experimental.pallas import tpu_sc as plsc

N = 10000
E = 320000
D_IN, D_HID, D_OUT = 128, 128, 64

NC, NS = 2, 16
NW = NC * NS
EPT = E // NW
CH = 80
NCHUNK = EPT // CH
RSTRIDE = 624
RWIN = 640


def _stripe(s):
    return pl.multiple_of(s * RSTRIDE, 8)

_mesh = functools.partial(
    plsc.VectorSubcoreMesh, core_axis_name="c", subcore_axis_name="s"
)

_SC_PARAMS = pltpu.CompilerParams(use_tc_tiling_on_sc=False,
                                  needs_layout_passes=False)


@functools.partial(
    pl.kernel,
    out_type=jax.ShapeDtypeStruct((NC, N, 8), jnp.float32),
    mesh=_mesh(),
    compiler_params=_SC_PARAMS,
    scratch_types=[
        pltpu.VMEM((NCHUNK, CH), jnp.int32),
        pltpu.VMEM((CH, 8), jnp.float32),
        pltpu.VMEM_SHARED((N, 8), jnp.float32),
    ],
)
def _deg_kernel(dst_hbm, ones_hbm, half_hbm, out_hbm, dst_v, ones_v, acc_sh):
    c = lax.axis_index("c")
    s = lax.axis_index("s")
    wid = s * NC + c
    pltpu.sync_copy(dst_hbm.at[wid], dst_v)
    pltpu.sync_copy(ones_hbm, ones_v)
    pltpu.sync_copy(half_hbm.at[pl.ds(_stripe(s), RWIN)],
                    acc_sh.at[pl.ds(_stripe(s), RWIN)])
    plsc.subcore_barrier()

    def body(j, carry):
        pltpu.sync_copy(ones_v, acc_sh.at[dst_v.at[j]], add=True)
        return carry

    lax.fori_loop(0, NCHUNK, body, 0)
    plsc.subcore_barrier()
    pltpu.sync_copy(
        acc_sh.at[pl.ds(_stripe(s), RWIN)], out_hbm.at[c, pl.ds(_stripe(s), RWIN)]
    )


CHS = 125
NCHS = EPT // CHS


def _make_segsum(D, KSEG):
    @functools.partial(
        pl.kernel,
        out_type=jax.ShapeDtypeStruct((NC, N, D), jnp.bfloat16),
        mesh=_mesh(),
        compiler_params=_SC_PARAMS,
        scratch_types=[
            pltpu.VMEM((NCHS, CHS), jnp.int32),
            pltpu.VMEM((NCHS, CHS), jnp.int32),
        ] + [pltpu.VMEM((CHS, D), jnp.bfloat16)] * KSEG
          + [pltpu.SemaphoreType.DMA] * KSEG
          + [pltpu.SemaphoreType.DMA,
             pltpu.VMEM_SHARED((N, D), jnp.bfloat16)],
    )
    def segsum(g_hbm, src_hbm, dst_hbm, out_hbm, src_v, dst_v, *scr):
        rows = scr[:KSEG]
        gsem = scr[KSEG:2 * KSEG]
        ssem = scr[2 * KSEG]
        acc_sh = scr[2 * KSEG + 1]
        c = lax.axis_index("c")
        s = lax.axis_index("s")
        wid = s * NC + c
        pltpu.sync_copy(src_hbm.at[wid], src_v)
        pltpu.sync_copy(dst_hbm.at[wid], dst_v)
        pltpu.sync_copy(g_hbm.at[pl.ds(_stripe(s), RWIN)],
                        acc_sh.at[pl.ds(_stripe(s), RWIN)])
        plsc.subcore_barrier()

        for p in range(KSEG):
            pltpu.async_copy(g_hbm.at[src_v.at[p]], rows[p], gsem[p])

        def body(i, carry):
            for b in range(KSEG):
                j = i * KSEG + b
                pltpu.make_async_copy(
                    g_hbm.at[src_v.at[j]], rows[b], gsem[b]).wait()
                cp = pltpu.async_copy(
                    rows[b], acc_sh.at[dst_v.at[j]], ssem, add=True)
                cp.wait()

                @pl.when(j + KSEG < NCHS)
                def _():
                    pltpu.async_copy(
                        g_hbm.at[src_v.at[j + KSEG]], rows[b], gsem[b])
            return carry

        lax.fori_loop(0, NCHS // KSEG, body, 0)
        plsc.subcore_barrier()
        pltpu.sync_copy(
            acc_sh.at[pl.ds(_stripe(s), RWIN)], out_hbm.at[c, pl.ds(_stripe(s), RWIN)]
        )

    return segsum


_segsum_hid = _make_segsum(D_HID, 4)
_segsum_out = _make_segsum(D_OUT, 4)


CHD = 125
NCHD = EPT // CHD
KDEC = 8


@functools.partial(
    pl.kernel,
    out_type=jax.ShapeDtypeStruct((NW, NCHD, CHD), jnp.float32),
    mesh=_mesh(),
    compiler_params=_SC_PARAMS,
    scratch_types=[
        pltpu.VMEM((NCHD, CHD), jnp.int32),
        pltpu.VMEM((NCHD, CHD), jnp.int32),
        pltpu.VMEM((NCHD, CHD), jnp.float32),
    ] + [pltpu.VMEM((CHD, D_OUT), jnp.bfloat16)] * (2 * KDEC)
      + [pltpu.SemaphoreType.DMA] * (2 * KDEC)
      + [pltpu.VMEM_SHARED((N, D_OUT), jnp.bfloat16)],
)
def _decode_kernel(z_hbm, src_hbm, dst_hbm, out_hbm, src_v, dst_v, sc_v, *scr):
    zs = scr[:KDEC]
    zd = scr[KDEC:2 * KDEC]
    sem_a = scr[2 * KDEC:3 * KDEC]
    sem_b = scr[3 * KDEC:4 * KDEC]
    z_sh = scr[4 * KDEC]
    c = lax.axis_index("c")
    s = lax.axis_index("s")
    wid = s * NC + c
    pltpu.sync_copy(src_hbm.at[wid], src_v)
    pltpu.sync_copy(dst_hbm.at[wid], dst_v)
    pltpu.sync_copy(z_hbm.at[pl.ds(_stripe(s), RWIN)],
                    z_sh.at[pl.ds(_stripe(s), RWIN)])
    plsc.subcore_barrier()

    for p in range(KDEC):
        pltpu.async_copy(z_sh.at[src_v.at[p]], zs[p], sem_a[p])
        pltpu.async_copy(z_sh.at[dst_v.at[p]], zd[p], sem_b[p])

    def body(i, carry):
        for bb in range(KDEC):
            j = i * KDEC + bb
            pltpu.make_async_copy(z_sh.at[src_v.at[j]], zs[bb], sem_a[bb]).wait()
            pltpu.make_async_copy(z_sh.at[dst_v.at[j]], zd[bb], sem_b[bb]).wait()
            lane = lax.iota(jnp.int32, 16)
            starts = list(range(0, CHD - 15, 16))
            if CHD % 16:
                starts.append(CHD - 16)
            for st in starts:
                res = jnp.zeros((16,), jnp.float32)
                for r16 in range(16):
                    r = st + r16
                    a0 = zs[bb][r, pl.ds(0, 32)]
                    b0 = zd[bb][r, pl.ds(0, 32)]
                    a1 = zs[bb][r, pl.ds(32, 32)]
                    b1 = zd[bb][r, pl.ds(32, 32)]
                    p16 = a0 * b0 + a1 * b1
                    u, v = plsc.unpack(p16, format=plsc.PackFormat.INTERLEAVED)
                    res = jnp.where(lane == r16, jnp.sum(u + v), res)
                sc_v[j, pl.ds(st, 16)] = 1.0 / (1.0 + jnp.exp(-res))

            @pl.when(j + KDEC < NCHD)
            def _():
                pltpu.async_copy(z_sh.at[src_v.at[j + KDEC]], zs[bb], sem_a[bb])
                pltpu.async_copy(z_sh.at[dst_v.at[j + KDEC]], zd[bb], sem_b[bb])
        return carry

    lax.fori_loop(0, NCHD // KDEC, body, 0)
    pltpu.sync_copy(sc_v, out_hbm.at[wid])


def _mm1_body(deg_ref, x_ref, w1_ref, g1_ref, dinv_ref):
    deg = deg_ref[0] + deg_ref[1]
    dinv = lax.rsqrt(deg[:, 0:1])
    g1_ref[...] = (dinv * jnp.dot(x_ref[...], w1_ref[...],
                                  preferred_element_type=jnp.float32)
                   ).astype(jnp.bfloat16)
    dinv_ref[...] = dinv


def _mm2_body(p_ref, g1_ref, dinv_ref, b1_ref, w2_ref, g2_ref):
    dinv = dinv_ref[...]
    agg = (p_ref[0].astype(jnp.float32) + p_ref[1].astype(jnp.float32)
           - g1_ref[...].astype(jnp.float32))
    h = jnp.maximum(dinv * agg + b1_ref[...], 0.0)
    g2_ref[...] = (dinv * jnp.dot(h, w2_ref[...],
                                  preferred_element_type=jnp.float32)
                   ).astype(jnp.bfloat16)


def _z_body(p_ref, g2_ref, dinv_ref, b2_ref, z_ref):
    agg = (p_ref[0].astype(jnp.float32) + p_ref[1].astype(jnp.float32)
           - g2_ref[...].astype(jnp.float32))
    z_ref[...] = (dinv_ref[...] * agg + b2_ref[...]).astype(jnp.bfloat16)


_BLK = 1000
_GRID = N // _BLK


def _mm1(degp, x, W1):
    return pl.pallas_call(
        _mm1_body,
        grid=(_GRID,),
        in_specs=[
            pl.BlockSpec((NC, _BLK, 8), lambda i: (0, i, 0)),
            pl.BlockSpec((_BLK, D_IN), lambda i: (i, 0)),
            pl.BlockSpec((D_IN, D_HID), lambda i: (0, 0)),
        ],
        out_specs=[
            pl.BlockSpec((_BLK, D_HID), lambda i: (i, 0)),
            pl.BlockSpec((_BLK, 1), lambda i: (i, 0)),
        ],
        out_shape=[
            jax.ShapeDtypeStruct((N, D_HID), jnp.bfloat16),
            jax.ShapeDtypeStruct((N, 1), jnp.float32),
        ],
    )(degp, x, W1)


def _mm2(p1, g1, dinv, b1, W2):
    return pl.pallas_call(
        _mm2_body,
        grid=(_GRID,),
        in_specs=[
            pl.BlockSpec((NC, _BLK, D_HID), lambda i: (0, i, 0)),
            pl.BlockSpec((_BLK, D_HID), lambda i: (i, 0)),
            pl.BlockSpec((_BLK, 1), lambda i: (i, 0)),
            pl.BlockSpec((1, D_HID), lambda i: (0, 0)),
            pl.BlockSpec((D_HID, D_OUT), lambda i: (0, 0)),
        ],
        out_specs=pl.BlockSpec((_BLK, D_OUT), lambda i: (i, 0)),
        out_shape=jax.ShapeDtypeStruct((N, D_OUT), jnp.bfloat16),
    )(p1, g1, dinv, b1, W2)


def _zstage(p2, g2, dinv, b2):
    return pl.pallas_call(
        _z_body,
        grid=(_GRID,),
        in_specs=[
            pl.BlockSpec((NC, _BLK, D_OUT), lambda i: (0, i, 0)),
            pl.BlockSpec((_BLK, D_OUT), lambda i: (i, 0)),
            pl.BlockSpec((_BLK, 1), lambda i: (i, 0)),
            pl.BlockSpec((1, D_OUT), lambda i: (0, 0)),
        ],
        out_specs=pl.BlockSpec((_BLK, D_OUT), lambda i: (i, 0)),
        out_shape=jax.ShapeDtypeStruct((N, D_OUT), jnp.bfloat16),
    )(p2, g2, dinv, b2)


def kernel(x, edge_index, W1, b1, W2, b2):
    dst_d = edge_index[1].reshape(NW, NCHUNK, CH)
    src_s = edge_index[0].reshape(NW, NCHS, CHS)
    dst_s = edge_index[1].reshape(NW, NCHS, CHS)
    src_c = edge_index[0].reshape(NW, NCHD, CHD)
    dst_c = edge_index[1].reshape(NW, NCHD, CHD)
    ones8 = jnp.ones((CH, 8), jnp.float32)
    half8 = jnp.full((N, 8), 0.5, jnp.float32)

    degp = _deg_kernel(dst_d, ones8, half8)
    g1, dinv = _mm1(degp, x, W1)
    p1 = _segsum_hid(g1, src_s, dst_s)
    g2 = _mm2(p1, g1, dinv, b1.reshape(1, D_HID), W2)
    p2 = _segsum_out(g2, src_s, dst_s)
    z = _zstage(p2, g2, dinv, b2.reshape(1, D_OUT))
    scores = _decode_kernel(z, src_c, dst_c)
    return scores.reshape(E, 1)


# speedup vs baseline: 1.0390x; 1.0390x over previous
import functools

import jax
import jax.numpy as jnp
from jax import lax
from jax.experimental import pallas as pl
from jax.experimental.pallas import tpu as pltpu
from jax.experimental.pallas import tpu_sc as plsc

N = 10000
E = 320000
D_IN, D_HID, D_OUT = 128, 128, 64

NC, NS = 2, 16
NW = NC * NS
EPT = E // NW
CH = 80
NCHUNK = EPT // CH
RSTRIDE = 624
RWIN = 640


def _stripe(s):
    return pl.multiple_of(s * RSTRIDE, 8)

_mesh = functools.partial(
    plsc.VectorSubcoreMesh, core_axis_name="c", subcore_axis_name="s"
)

_SC_PARAMS = pltpu.CompilerParams(use_tc_tiling_on_sc=False,
                                  needs_layout_passes=False)


@functools.partial(
    pl.kernel,
    out_type=jax.ShapeDtypeStruct((NC, N, 8), jnp.float32),
    mesh=_mesh(),
    compiler_params=_SC_PARAMS,
    scratch_types=[
        pltpu.VMEM((NCHUNK, CH), jnp.int32),
        pltpu.VMEM((CH, 8), jnp.float32),
        pltpu.VMEM_SHARED((N, 8), jnp.float32),
    ],
)
def _deg_kernel(dst_hbm, ones_hbm, half_hbm, out_hbm, dst_v, ones_v, acc_sh):
    c = lax.axis_index("c")
    s = lax.axis_index("s")
    wid = s * NC + c
    pltpu.sync_copy(dst_hbm.at[wid], dst_v)
    pltpu.sync_copy(ones_hbm, ones_v)
    pltpu.sync_copy(half_hbm.at[pl.ds(_stripe(s), RWIN)],
                    acc_sh.at[pl.ds(_stripe(s), RWIN)])
    plsc.subcore_barrier()

    def body(j, carry):
        pltpu.sync_copy(ones_v, acc_sh.at[dst_v.at[j]], add=True)
        return carry

    lax.fori_loop(0, NCHUNK, body, 0)
    plsc.subcore_barrier()
    pltpu.sync_copy(
        acc_sh.at[pl.ds(_stripe(s), RWIN)], out_hbm.at[c, pl.ds(_stripe(s), RWIN)]
    )


CHS = 125
NCHS = EPT // CHS


def _make_segsum(D, KSEG):
    @functools.partial(
        pl.kernel,
        out_type=jax.ShapeDtypeStruct((NC, N, D), jnp.bfloat16),
        mesh=_mesh(),
        compiler_params=_SC_PARAMS,
        scratch_types=[
            pltpu.VMEM((NCHS, CHS), jnp.int32),
            pltpu.VMEM((NCHS, CHS), jnp.int32),
        ] + [pltpu.VMEM((CHS, D), jnp.bfloat16)] * KSEG
          + [pltpu.SemaphoreType.DMA] * KSEG
          + [pltpu.SemaphoreType.DMA,
             pltpu.VMEM_SHARED((N, D), jnp.bfloat16)],
    )
    def segsum(g_hbm, src_hbm, dst_hbm, out_hbm, src_v, dst_v, *scr):
        rows = scr[:KSEG]
        gsem = scr[KSEG:2 * KSEG]
        ssem = scr[2 * KSEG]
        acc_sh = scr[2 * KSEG + 1]
        c = lax.axis_index("c")
        s = lax.axis_index("s")
        wid = s * NC + c
        pltpu.sync_copy(src_hbm.at[wid], src_v)
        pltpu.sync_copy(dst_hbm.at[wid], dst_v)
        pltpu.sync_copy(g_hbm.at[pl.ds(_stripe(s), RWIN)],
                        acc_sh.at[pl.ds(_stripe(s), RWIN)])
        plsc.subcore_barrier()

        for p in range(KSEG):
            pltpu.async_copy(g_hbm.at[src_v.at[p]], rows[p], gsem[p])

        def body(i, carry):
            for b in range(KSEG):
                j = i * KSEG + b
                pltpu.make_async_copy(
                    g_hbm.at[src_v.at[j]], rows[b], gsem[b]).wait()
                cp = pltpu.async_copy(
                    rows[b], acc_sh.at[dst_v.at[j]], ssem, add=True)
                cp.wait()

                @pl.when(j + KSEG < NCHS)
                def _():
                    pltpu.async_copy(
                        g_hbm.at[src_v.at[j + KSEG]], rows[b], gsem[b])
            return carry

        lax.fori_loop(0, NCHS // KSEG, body, 0)
        plsc.subcore_barrier()
        pltpu.sync_copy(
            acc_sh.at[pl.ds(_stripe(s), RWIN)], out_hbm.at[c, pl.ds(_stripe(s), RWIN)]
        )

    return segsum


_segsum_hid = _make_segsum(D_HID, 4)
_segsum_out = _make_segsum(D_OUT, 4)


CHD = 125
NCHD = EPT // CHD
KDEC = 5


@functools.partial(
    pl.kernel,
    out_type=jax.ShapeDtypeStruct((NW, NCHD, CHD), jnp.float32),
    mesh=_mesh(),
    compiler_params=_SC_PARAMS,
    scratch_types=[
        pltpu.VMEM((NCHD, CHD), jnp.int32),
        pltpu.VMEM((NCHD, CHD), jnp.int32),
        pltpu.VMEM((NCHD, CHD), jnp.float32),
    ] + [pltpu.VMEM((CHD, D_OUT), jnp.bfloat16)] * (2 * KDEC)
      + [pltpu.SemaphoreType.DMA] * (2 * KDEC)
      + [pltpu.VMEM_SHARED((N, D_OUT), jnp.bfloat16)],
)
def _decode_kernel(z_hbm, src_hbm, dst_hbm, out_hbm, src_v, dst_v, sc_v, *scr):
    zs = scr[:KDEC]
    zd = scr[KDEC:2 * KDEC]
    sem_a = scr[2 * KDEC:3 * KDEC]
    sem_b = scr[3 * KDEC:4 * KDEC]
    z_sh = scr[4 * KDEC]
    c = lax.axis_index("c")
    s = lax.axis_index("s")
    wid = s * NC + c
    pltpu.sync_copy(src_hbm.at[wid], src_v)
    pltpu.sync_copy(dst_hbm.at[wid], dst_v)
    pltpu.sync_copy(z_hbm.at[pl.ds(_stripe(s), RWIN)],
                    z_sh.at[pl.ds(_stripe(s), RWIN)])
    plsc.subcore_barrier()

    for p in range(KDEC):
        pltpu.async_copy(z_sh.at[src_v.at[p]], zs[p], sem_a[p])
        pltpu.async_copy(z_sh.at[dst_v.at[p]], zd[p], sem_b[p])

    def body(i, carry):
        for bb in range(KDEC):
            j = i * KDEC + bb
            pltpu.make_async_copy(z_sh.at[src_v.at[j]], zs[bb], sem_a[bb]).wait()
            pltpu.make_async_copy(z_sh.at[dst_v.at[j]], zd[bb], sem_b[bb]).wait()
            lane = lax.iota(jnp.int32, 16)
            starts = list(range(0, CHD - 15, 16))
            if CHD % 16:
                starts.append(CHD - 16)
            for st in starts:
                res = jnp.zeros((16,), jnp.float32)
                for r16 in range(16):
                    r = st + r16
                    a0 = zs[bb][r, pl.ds(0, 32)]
                    b0 = zd[bb][r, pl.ds(0, 32)]
                    a1 = zs[bb][r, pl.ds(32, 32)]
                    b1 = zd[bb][r, pl.ds(32, 32)]
                    p16 = a0 * b0 + a1 * b1
                    u, v = plsc.unpack(p16, format=plsc.PackFormat.INTERLEAVED)
                    res = jnp.where(lane == r16, jnp.sum(u + v), res)
                sc_v[j, pl.ds(st, 16)] = 1.0 / (1.0 + jnp.exp(-res))

            @pl.when(j + KDEC < NCHD)
            def _():
                pltpu.async_copy(z_sh.at[src_v.at[j + KDEC]], zs[bb], sem_a[bb])
                pltpu.async_copy(z_sh.at[dst_v.at[j + KDEC]], zd[bb], sem_b[bb])
        return carry

    lax.fori_loop(0, NCHD // KDEC, body, 0)
    pltpu.sync_copy(sc_v, out_hbm.at[wid])


def _mm1_body(deg_ref, x_ref, w1_ref, g1_ref, dinv_ref):
    deg = deg_ref[0] + deg_ref[1]
    dinv = lax.rsqrt(deg[:, 0:1])
    g1_ref[...] = (dinv * jnp.dot(x_ref[...], w1_ref[...],
                                  preferred_element_type=jnp.float32)
                   ).astype(jnp.bfloat16)
    dinv_ref[...] = dinv


def _mm2_body(p_ref, g1_ref, dinv_ref, b1_ref, w2_ref, g2_ref):
    dinv = dinv_ref[...]
    agg = (p_ref[0].astype(jnp.float32) + p_ref[1].astype(jnp.float32)
           - g1_ref[...].astype(jnp.float32))
    h = jnp.maximum(dinv * agg + b1_ref[...], 0.0)
    g2_ref[...] = (dinv * jnp.dot(h, w2_ref[...],
                                  preferred_element_type=jnp.float32)
                   ).astype(jnp.bfloat16)


def _z_body(p_ref, g2_ref, dinv_ref, b2_ref, z_ref):
    agg = (p_ref[0].astype(jnp.float32) + p_ref[1].astype(jnp.float32)
           - g2_ref[...].astype(jnp.float32))
    z_ref[...] = (dinv_ref[...] * agg + b2_ref[...]).astype(jnp.bfloat16)


_BLK = 1000
_GRID = N // _BLK


def _mm1(degp, x, W1):
    return pl.pallas_call(
        _mm1_body,
        grid=(_GRID,),
        in_specs=[
            pl.BlockSpec((NC, _BLK, 8), lambda i: (0, i, 0)),
            pl.BlockSpec((_BLK, D_IN), lambda i: (i, 0)),
            pl.BlockSpec((D_IN, D_HID), lambda i: (0, 0)),
        ],
        out_specs=[
            pl.BlockSpec((_BLK, D_HID), lambda i: (i, 0)),
            pl.BlockSpec((_BLK, 1), lambda i: (i, 0)),
        ],
        out_shape=[
            jax.ShapeDtypeStruct((N, D_HID), jnp.bfloat16),
            jax.ShapeDtypeStruct((N, 1), jnp.float32),
        ],
    )(degp, x, W1)


def _mm2(p1, g1, dinv, b1, W2):
    return pl.pallas_call(
        _mm2_body,
        grid=(_GRID,),
        in_specs=[
            pl.BlockSpec((NC, _BLK, D_HID), lambda i: (0, i, 0)),
            pl.BlockSpec((_BLK, D_HID), lambda i: (i, 0)),
            pl.BlockSpec((_BLK, 1), lambda i: (i, 0)),
            pl.BlockSpec((1, D_HID), lambda i: (0, 0)),
            pl.BlockSpec((D_HID, D_OUT), lambda i: (0, 0)),
        ],
        out_specs=pl.BlockSpec((_BLK, D_OUT), lambda i: (i, 0)),
        out_shape=jax.ShapeDtypeStruct((N, D_OUT), jnp.bfloat16),
    )(p1, g1, dinv, b1, W2)


def _zstage(p2, g2, dinv, b2):
    return pl.pallas_call(
        _z_body,
        grid=(_GRID,),
        in_specs=[
            pl.BlockSpec((NC, _BLK, D_OUT), lambda i: (0, i, 0)),
            pl.BlockSpec((_BLK, D_OUT), lambda i: (i, 0)),
            pl.BlockSpec((_BLK, 1), lambda i: (i, 0)),
            pl.BlockSpec((1, D_OUT), lambda i: (0, 0)),
        ],
        out_specs=pl.BlockSpec((_BLK, D_OUT), lambda i: (i, 0)),
        out_shape=jax.ShapeDtypeStruct((N, D_OUT), jnp.bfloat16),
    )(p2, g2, dinv, b2)


def kernel(x, edge_index, W1, b1, W2, b2):
    dst_d = edge_index[1].reshape(NW, NCHUNK, CH)
    src_s = edge_index[0].reshape(NW, NCHS, CHS)
    dst_s = edge_index[1].reshape(NW, NCHS, CHS)
    src_c = edge_index[0].reshape(NW, NCHD, CHD)
    dst_c = edge_index[1].reshape(NW, NCHD, CHD)
    ones8 = jnp.ones((CH, 8), jnp.float32)
    half8 = jnp.full((N, 8), 0.5, jnp.float32)

    degp = _deg_kernel(dst_d, ones8, half8)
    g1, dinv = _mm1(degp, x, W1)
    p1 = _segsum_hid(g1, src_s, dst_s)
    g2 = _mm2(p1, g1, dinv, b1.reshape(1, D_HID), W2)
    p2 = _segsum_out(g2, src_s, dst_s)
    z = _zstage(p2, g2, dinv, b2.reshape(1, D_OUT))
    scores = _decode_kernel(z, src_c, dst_c)
    return scores.reshape(E, 1)
